# Initial kernel scaffold; baseline (speedup 1.0000x reference)
import jax
import jax.numpy as jnp
from jax.experimental import pallas as pl


def kernel(output, anchors, targets):
    raise NotImplementedError("write your pallas kernel here")


# speedup vs baseline: 11.8023x; 11.8023x over previous
import jax
import jax.numpy as jnp
from jax.experimental import pallas as pl

_H = 52
_W = 52
_A = 9
_FLAT = _H * _W * _A
_THRESHOLD = 0.5
_NOOBJ_W = 0.5


def _obj_loss_kernel(pred_ref, tgt_ref, anc_ref, out_ref):
    b = pred_ref.shape[0]
    p = pred_ref[:, :]
    logp = jnp.maximum(jnp.log(p), -100.0)
    log1mp = jnp.maximum(jnp.log(1.0 - p), -100.0)

    tx = tgt_ref[1]
    ty = tgt_ref[2]
    tw = tgt_ref[3]
    th = tgt_ref[4]
    keep = jnp.logical_not((tx == 0.0) & (ty == 0.0) & (tw == 0.0) & (th == 0.0))
    cx = jnp.floor(tx * _W)
    cy = jnp.floor(ty * _H)
    t0 = (tx - (cx + 0.5) / _W) * _W
    t1 = (ty - (cy + 0.5) / _H) * _H
    t2 = tw * _W
    t3 = th * _H

    aw = anc_ref[0]
    ah = anc_ref[1]
    tx0 = (t0 - t2 / 2)[:, :, None]
    ty0 = (t1 - t3 / 2)[:, :, None]
    tx1 = (t0 + t2 / 2)[:, :, None]
    ty1 = (t1 + t3 / 2)[:, :, None]
    ax0 = (-aw / 2)[None, None, :]
    ay0 = (-ah / 2)[None, None, :]
    ax1 = (aw / 2)[None, None, :]
    ay1 = (ah / 2)[None, None, :]
    x0 = jnp.maximum(tx0, ax0)
    y0 = jnp.maximum(ty0, ay0)
    x1 = jnp.minimum(tx1, ax1)
    y1 = jnp.minimum(ty1, ay1)
    flag = ((x0 < x1) & (y0 < y1)).astype(jnp.float32)
    inter = (x1 - x0) * (y1 - y0) * flag
    a_area = (aw * ah)[None, None, :]
    t_area = (t2 * t3)[:, :, None]
    ious = inter / (t_area + a_area - inter)

    maxv = jnp.max(ious, axis=2, keepdims=True)
    aiota = jax.lax.broadcasted_iota(jnp.int32, (b, _A, _A), 2).astype(jnp.float32)
    aidx = jnp.min(jnp.where(ious == maxv, aiota, float(_A)), axis=2)
    mask = maxv[:, :, 0] > _THRESHOLD
    cell = cy * _W + cx

    kf = jax.lax.broadcasted_iota(jnp.int32, (b, _FLAT), 1).astype(jnp.float32)
    nf = jnp.floor(kf * (1.0 / _A))
    af = kf - nf * _A

    gt = jnp.zeros((b, _FLAT), jnp.float32)
    for i in range(_A):
        cm = (nf == cell[:, i : i + 1]) & keep[:, i : i + 1]
        vi = jnp.where((af == aidx[:, i : i + 1]) & mask[:, i : i + 1], 1.0, -100.0)
        gt = jnp.where(cm, vi, gt)

    contrib = jnp.where(
        gt == 1.0, -logp, jnp.where(gt == 0.0, -_NOOBJ_W * log1mp, 0.0)
    )
    row = jnp.sum(contrib, axis=1, keepdims=True)
    total = jnp.sum(row, axis=0, keepdims=True)
    out_ref[:, :] = total * (1.0 / _FLAT) * (1.0 / b)


def kernel(output, anchors, targets):
    b, a, h, w, _ = output.shape
    pred = output[..., 4].reshape(b, a * h * w)
    tgt_t = jnp.transpose(targets, (2, 0, 1))
    anc_t = jnp.transpose(anchors, (1, 0))
    out = pl.pallas_call(
        _obj_loss_kernel,
        out_shape=jax.ShapeDtypeStruct((1, 1), jnp.float32),
    )(pred, tgt_t, anc_t)
    return out[0, 0]
